# baseline (speedup 1.0000x reference)
import functools

import jax
import jax.numpy as jnp
from jax import lax
from jax.experimental import pallas as pl
from jax.experimental.pallas import tpu as pltpu
from jax.experimental.pallas import tpu_sc as plsc

_VOCAB = 100000
_HIDDEN = 768
_MAX_POS = 2048
_BATCH = 4
_SEQ = 2048

_L = 16
_NV = _HIDDEN // _L
_NW = 32
_S_PER_W = _SEQ // _NW
_CH = 16
_NBUF = 4
_NCHUNK = (_S_PER_W // _CH) * _BATCH
_NGRP = _NCHUNK // _NBUF
_ROW_UNROLL = 2
_INV_H = 1.0 / _HIDDEN
_EPS = 1e-12


def _lane_sum(v):
    dnums = lax.GatherDimensionNumbers(
        offset_dims=(), collapsed_slice_dims=(0,), start_index_map=(0,))
    for sh in (8, 4, 2, 1):
        idx = lax.iota(jnp.int32, _L) ^ sh
        v = v + lax.gather(v, idx[:, None], dnums, (1,),
                           mode=lax.GatherScatterMode.PROMISE_IN_BOUNDS)
    return v


def _rsqrt_vec(v):
    i = lax.bitcast_convert_type(v, jnp.int32)
    i = jnp.full((_L,), 0x5F3759DF, jnp.int32) - lax.shift_right_logical(
        i, jnp.full((_L,), 1, jnp.int32))
    y = lax.bitcast_convert_type(i, jnp.float32)
    half_v = v * 0.5
    for _ in range(3):
        y = y * (1.5 - half_v * y * y)
    return y


def _emb_body(x_hbm, pos_hbm, gamma_hbm, beta_hbm, tok_hbm, out_hbm,
              idx_v, pos_v, tok0_v, tok1_v, tok2_v, tok3_v,
              gamma_v, beta_v,
              gsem0, gsem1, gsem2, gsem3,
              ssem0, ssem1, ssem2, ssem3, stage_sem):
    nc = 2
    wid = lax.axis_index("s") * nc + lax.axis_index("c")
    s0w = wid * _S_PER_W

    bufs = (tok0_v, tok1_v, tok2_v, tok3_v)
    gsems = (gsem0, gsem1, gsem2, gsem3)
    ssems = (ssem0, ssem1, ssem2, ssem3)

    stages = [(gamma_hbm, gamma_v), (beta_hbm, beta_v),
              (pos_hbm.at[pl.ds(s0w, _S_PER_W)], pos_v)]
    for c in range(_NCHUNK):
        q, b = c // _BATCH, c % _BATCH
        stages.append((x_hbm.at[b, pl.ds(s0w + q * _CH, _CH)], idx_v.at[c]))
    handles = [pltpu.async_copy(src, dst, stage_sem) for src, dst in stages]
    for h in handles:
        h.wait()

    def norm_row(buf, pos_base, r):
        acc_s = jnp.zeros((_L,), jnp.float32)
        acc_q = jnp.zeros((_L,), jnp.float32)
        for j in range(_NV):
            t = buf[r, pl.ds(j * _L, _L)] + pos_v[pos_base + r,
                                                  pl.ds(j * _L, _L)]
            buf[r, pl.ds(j * _L, _L)] = t
            acc_s = acc_s + t
            acc_q = acc_q + t * t
        mean = _lane_sum(acc_s) * _INV_H
        ex2 = _lane_sum(acc_q) * _INV_H
        var = ex2 - mean * mean
        rstd = _rsqrt_vec(var + _EPS)
        for j in range(_NV):
            t = buf[r, pl.ds(j * _L, _L)]
            o = (t - mean) * rstd * gamma_v[pl.ds(j * _L, _L)] \
                + beta_v[pl.ds(j * _L, _L)]
            buf[r, pl.ds(j * _L, _L)] = o

    def compute_chunk(buf, pos_base):
        def row_body(r, _):
            for u in range(_ROW_UNROLL):
                norm_row(buf, pos_base, r * _ROW_UNROLL + u)
            return 0
        lax.fori_loop(0, _CH // _ROW_UNROLL, row_body, 0, unroll=False)

    def gather(c, k):
        return pltpu.async_copy(tok_hbm.at[idx_v.at[c]], bufs[k], gsems[k])

    def wait_gather(c, k):
        pltpu.make_async_copy(tok_hbm.at[idx_v.at[c]], bufs[k],
                              gsems[k]).wait()

    def wait_store(k):
        pltpu.make_async_copy(bufs[k], out_hbm.at[0, pl.ds(0, _CH)],
                              ssems[k]).wait()

    for c in range(_NBUF - 1):
        gather(c, c)

    def group(i, _):
        for k in range(_NBUF):
            c = _NBUF * i + k
            wait_gather(c, k)
            compute_chunk(bufs[k], i * _CH)
            if k == 0:
                @pl.when(i > 0)
                def _():
                    wait_store((k - 1) % _NBUF)
                gather(c + _NBUF - 1, (k - 1) % _NBUF)
            else:
                wait_store(k - 1)

                @pl.when(i < _NGRP - 1)
                def _():
                    gather(c + _NBUF - 1, k - 1)
            pltpu.async_copy(
                bufs[k], out_hbm.at[k, pl.ds(s0w + i * _CH, _CH)], ssems[k])
        return 0

    lax.fori_loop(0, _NGRP, group, 0, unroll=False)
    wait_store(_NBUF - 1)


@jax.jit
def kernel(x, token_table, pos_table, gamma, beta):
    mesh = plsc.VectorSubcoreMesh(core_axis_name="c", subcore_axis_name="s")
    run = functools.partial(
        pl.kernel,
        mesh=mesh,
        out_type=jax.ShapeDtypeStruct((_BATCH, _SEQ, _HIDDEN), jnp.float32),
        scratch_types=[
            pltpu.VMEM((_NCHUNK, _CH), jnp.int32),
            pltpu.VMEM((_S_PER_W, _HIDDEN), jnp.float32),
            pltpu.VMEM((_CH, _HIDDEN), jnp.float32),
            pltpu.VMEM((_CH, _HIDDEN), jnp.float32),
            pltpu.VMEM((_CH, _HIDDEN), jnp.float32),
            pltpu.VMEM((_CH, _HIDDEN), jnp.float32),
            pltpu.VMEM((_HIDDEN,), jnp.float32),
            pltpu.VMEM((_HIDDEN,), jnp.float32),
            pltpu.SemaphoreType.DMA,
            pltpu.SemaphoreType.DMA,
            pltpu.SemaphoreType.DMA,
            pltpu.SemaphoreType.DMA,
            pltpu.SemaphoreType.DMA,
            pltpu.SemaphoreType.DMA,
            pltpu.SemaphoreType.DMA,
            pltpu.SemaphoreType.DMA,
            pltpu.SemaphoreType.DMA,
        ],
    )(_emb_body)
    return run(x, pos_table, gamma, beta, token_table)


# speedup vs baseline: 2.3549x; 2.3549x over previous
import functools

import jax
import jax.numpy as jnp
from jax import lax
from jax.experimental import pallas as pl
from jax.experimental.pallas import tpu as pltpu
from jax.experimental.pallas import tpu_sc as plsc

_VOCAB = 100000
_HIDDEN = 768
_MAX_POS = 2048
_BATCH = 4
_SEQ = 2048

_NROW = _BATCH * _SEQ
_NW = 32
_R_PER_W = _NROW // _NW
_GCH = 64
_NGCH = _R_PER_W // _GCH
_EPS = 1e-12

_LN_ROWS = 256
_LN_GRID = _NROW // _LN_ROWS
_SEQ_BLKS = _SEQ // _LN_ROWS


def _gather_body(x_hbm, tok_hbm, out_hbm, idx_v, buf0, buf1,
                 gsem0, gsem1, ssem0, ssem1):
    nc = 2
    wid = lax.axis_index("s") * nc + lax.axis_index("c")
    r0 = wid * _R_PER_W
    pltpu.sync_copy(x_hbm.at[pl.ds(r0, _R_PER_W)], idx_v)

    bufs = (buf0, buf1)
    gsems = (gsem0, gsem1)
    ssems = (ssem0, ssem1)

    def gather(j):
        p = j % 2
        return pltpu.async_copy(
            tok_hbm.at[idx_v.at[pl.ds(j * _GCH, _GCH)]], bufs[p], gsems[p])

    def store(j):
        p = j % 2
        return pltpu.async_copy(
            bufs[p], out_hbm.at[pl.ds(r0 + j * _GCH, _GCH)], ssems[p])

    gh = {0: gather(0), 1: gather(1)}
    sh = {}
    for j in range(_NGCH):
        gh[j].wait()
        sh[j] = store(j)
        if j >= 1 and j + 1 < _NGCH:
            sh[j - 1].wait()
            gh[j + 1] = gather(j + 1)
    sh[_NGCH - 2].wait()
    sh[_NGCH - 1].wait()


def _sc_gather(x_flat, token_table):
    mesh = plsc.VectorSubcoreMesh(core_axis_name="c", subcore_axis_name="s")
    run = functools.partial(
        pl.kernel,
        mesh=mesh,
        out_type=jax.ShapeDtypeStruct((_NROW, _HIDDEN), jnp.float32),
        scratch_types=[
            pltpu.VMEM((_R_PER_W,), jnp.int32),
            pltpu.VMEM((_GCH, _HIDDEN), jnp.float32),
            pltpu.VMEM((_GCH, _HIDDEN), jnp.float32),
            pltpu.SemaphoreType.DMA,
            pltpu.SemaphoreType.DMA,
            pltpu.SemaphoreType.DMA,
            pltpu.SemaphoreType.DMA,
        ],
    )(_gather_body)
    return run(x_flat, token_table)


def _ln_body(tok_ref, pos_ref, gamma_ref, beta_ref, out_ref):
    h = tok_ref[...] + pos_ref[...]
    mean = jnp.mean(h, axis=-1, keepdims=True)
    d = h - mean
    var = jnp.mean(d * d, axis=-1, keepdims=True)
    out_ref[...] = d * lax.rsqrt(var + _EPS) * gamma_ref[...] \
        + beta_ref[...]


def _tc_layernorm(tok, pos_table, gamma, beta):
    return pl.pallas_call(
        _ln_body,
        grid=(_LN_GRID,),
        in_specs=[
            pl.BlockSpec((_LN_ROWS, _HIDDEN), lambda i: (i, 0)),
            pl.BlockSpec((_LN_ROWS, _HIDDEN), lambda i: (i % _SEQ_BLKS, 0)),
            pl.BlockSpec((_HIDDEN,), lambda i: (0,)),
            pl.BlockSpec((_HIDDEN,), lambda i: (0,)),
        ],
        out_specs=pl.BlockSpec((_LN_ROWS, _HIDDEN), lambda i: (i, 0)),
        out_shape=jax.ShapeDtypeStruct((_NROW, _HIDDEN), jnp.float32),
    )(tok, pos_table, gamma, beta)


@jax.jit
def kernel(x, token_table, pos_table, gamma, beta):
    tok = _sc_gather(x.reshape(-1), token_table)
    out = _tc_layernorm(tok, pos_table, gamma, beta)
    return out.reshape(_BATCH, _SEQ, _HIDDEN)


# speedup vs baseline: 2.9479x; 1.2518x over previous
import functools

import jax
import jax.numpy as jnp
from jax import lax
from jax.experimental import pallas as pl
from jax.experimental.pallas import tpu as pltpu
from jax.experimental.pallas import tpu_sc as plsc

_VOCAB = 100000
_HIDDEN = 768
_MAX_POS = 2048
_BATCH = 4
_SEQ = 2048

_NROW = _BATCH * _SEQ
_NW = 32
_R_PER_W = _NROW // _NW
_GCH = 64
_NGCH = _R_PER_W // _GCH
_EPS = 1e-12

_LN_ROWS = 256
_LN_GRID = _SEQ // _LN_ROWS


def _gather_body(x_hbm, tok_hbm, out_hbm, idx_v, buf0, buf1,
                 gsem0, gsem1, ssem0, ssem1):
    nc = 2
    wid = lax.axis_index("s") * nc + lax.axis_index("c")
    r0 = wid * _R_PER_W
    pltpu.sync_copy(x_hbm.at[pl.ds(r0, _R_PER_W)], idx_v)

    bufs = (buf0, buf1)
    gsems = (gsem0, gsem1)
    ssems = (ssem0, ssem1)

    def gather(j):
        p = j % 2
        return pltpu.async_copy(
            tok_hbm.at[idx_v.at[pl.ds(j * _GCH, _GCH)]], bufs[p], gsems[p])

    def store(j):
        p = j % 2
        return pltpu.async_copy(
            bufs[p], out_hbm.at[pl.ds(r0 + j * _GCH, _GCH)], ssems[p])

    gh = {0: gather(0), 1: gather(1)}
    sh = {}
    for j in range(_NGCH):
        gh[j].wait()
        sh[j] = store(j)
        if j >= 1 and j + 1 < _NGCH:
            sh[j - 1].wait()
            gh[j + 1] = gather(j + 1)
    sh[_NGCH - 2].wait()
    sh[_NGCH - 1].wait()


def _sc_gather(x_flat, token_table):
    mesh = plsc.VectorSubcoreMesh(core_axis_name="c", subcore_axis_name="s")
    run = functools.partial(
        pl.kernel,
        mesh=mesh,
        out_type=jax.ShapeDtypeStruct((_NROW, _HIDDEN), jnp.float32),
        scratch_types=[
            pltpu.VMEM((_R_PER_W,), jnp.int32),
            pltpu.VMEM((_GCH, _HIDDEN), jnp.float32),
            pltpu.VMEM((_GCH, _HIDDEN), jnp.float32),
            pltpu.SemaphoreType.DMA,
            pltpu.SemaphoreType.DMA,
            pltpu.SemaphoreType.DMA,
            pltpu.SemaphoreType.DMA,
        ],
    )(_gather_body)
    return run(x_flat, token_table)


def _ln_body(tok_ref, pos_ref, gamma_ref, beta_ref, out_ref):
    h = tok_ref[...] + pos_ref[...][None]
    mean = jnp.mean(h, axis=-1, keepdims=True)
    d = h - mean
    var = jnp.mean(d * d, axis=-1, keepdims=True)
    out_ref[...] = d * lax.rsqrt(var + _EPS) * gamma_ref[...] \
        + beta_ref[...]


def _tc_layernorm(tok, pos_table, gamma, beta):
    return pl.pallas_call(
        _ln_body,
        grid=(_LN_GRID,),
        in_specs=[
            pl.BlockSpec((_BATCH, _LN_ROWS, _HIDDEN), lambda i: (0, i, 0)),
            pl.BlockSpec((_LN_ROWS, _HIDDEN), lambda i: (i, 0)),
            pl.BlockSpec((_HIDDEN,), lambda i: (0,)),
            pl.BlockSpec((_HIDDEN,), lambda i: (0,)),
        ],
        out_specs=pl.BlockSpec((_BATCH, _LN_ROWS, _HIDDEN),
                               lambda i: (0, i, 0)),
        out_shape=jax.ShapeDtypeStruct((_BATCH, _SEQ, _HIDDEN), jnp.float32),
    )(tok, pos_table, gamma, beta)


@jax.jit
def kernel(x, token_table, pos_table, gamma, beta):
    tok = _sc_gather(x.reshape(-1), token_table)
    return _tc_layernorm(tok.reshape(_BATCH, _SEQ, _HIDDEN),
                         pos_table, gamma, beta)
